# Initial kernel scaffold; baseline (speedup 1.0000x reference)
import jax
import jax.numpy as jnp
from jax.experimental import pallas as pl


def kernel(x, edge_index, W0, U1_W, bn1_g, bn1_b, U2_W, bn2_g, bn2_b, U3_W, bn3_g, bn3_b, ro_W0, ro_b0, ro_W1, ro_b1, ro_W2, ro_b2, ro_W3, ro_b3):
    raise NotImplementedError("write your pallas kernel here")


# speedup vs baseline: 23.1284x; 23.1284x over previous
import functools

import jax
import jax.numpy as jnp
from jax import lax
from jax.experimental import pallas as pl
from jax.experimental.pallas import tpu as pltpu
from jax.experimental.pallas import tpu_sc as plsc

N = 10000
NP = 10240
E = 320000
D = 128
H = 128
OUT = 40

NC = 2
NS = 16
NW = NC * NS
K = 128
C = 80
EP = NW * C * K
T = C // 2
RS = NP // NS
BN = 2048
GRID = NP // BN

_SC_MESH = plsc.VectorSubcoreMesh(core_axis_name="c", subcore_axis_name="s")

_BN_SCALE = 1.0 / (1.0 + 1e-5) ** 0.5


@functools.partial(
    pl.kernel,
    out_type=jax.ShapeDtypeStruct((NC, NP, 16), jnp.float32),
    mesh=_SC_MESH,
    scratch_types=[
        pltpu.VMEM((C, K), jnp.int32),
        pltpu.VMEM((K, 16), jnp.float32),
        pltpu.VMEM_SHARED((NP, 16), jnp.float32),
        pltpu.SemaphoreType.DMA,
    ],
)
def _deg_kernel(row_hbm, out_hbm, rowv, onev, deg_sh, sem):
    c = lax.axis_index("c")
    s = lax.axis_index("s")
    w = c * NS + s
    pltpu.sync_copy(row_hbm.at[w], rowv)

    zero = jnp.zeros((16,), jnp.float32)

    def _zb(i, carry):
        onev[i, :] = zero
        return carry

    lax.fori_loop(0, K, _zb, 0)
    for q in range(RS // K):
        pltpu.sync_copy(onev, deg_sh.at[pl.ds(s * RS + q * K, K)])

    one = jnp.full((16,), 1.0, jnp.float32)

    def _ob(i, carry):
        onev[i, :] = one
        return carry

    lax.fori_loop(0, K, _ob, 0)
    plsc.subcore_barrier()

    def _fire(j):
        pltpu.async_copy(onev, deg_sh.at[rowv.at[j]], sem, add=True)

    for j in range(4):
        _fire(j)

    def _body(j, carry):
        @pl.when(j + 4 < C)
        def _():
            _fire(j + 4)

        pltpu.make_async_copy(onev, deg_sh.at[rowv.at[0]], sem).wait()
        return carry

    lax.fori_loop(0, C, _body, 0)
    plsc.subcore_barrier()
    pltpu.sync_copy(deg_sh.at[pl.ds(s * RS, RS)],
                    out_hbm.at[c, pl.ds(s * RS, RS)])


@functools.partial(
    pl.kernel,
    out_type=jax.ShapeDtypeStruct((NC, NP, H), jnp.float32),
    mesh=_SC_MESH,
    scratch_types=[
        pltpu.VMEM((C, K), jnp.int32),
        pltpu.VMEM((2, K), jnp.int32),
        pltpu.VMEM((2, K), jnp.int32),
        pltpu.VMEM((2 * K, H), jnp.float32),
        pltpu.VMEM_SHARED((NP, H), jnp.float32),
        pltpu.SemaphoreType.DMA,
        pltpu.SemaphoreType.DMA,
    ],
)
def _agg_kernel(hs_hbm, pk_hbm, out_hbm, pkv, colb, rowb, buf, agg_sh,
                gsem, ssem):
    c = lax.axis_index("c")
    s = lax.axis_index("s")
    w = c * NS + s
    pltpu.sync_copy(pk_hbm.at[w], pkv)

    rs = s * RS

    @pl.when(c == 0)
    def _():
        pltpu.sync_copy(hs_hbm.at[pl.ds(rs, RS)], agg_sh.at[pl.ds(rs, RS)])

    @pl.when(c == 1)
    def _():
        zero = jnp.zeros((16,), jnp.float32)

        def _zb(i, carry):
            r = i // (H // 16)
            l = i % (H // 16)
            buf[r, pl.ds(l * 16, 16)] = zero
            return carry

        lax.fori_loop(0, 2 * K * (H // 16), _zb, 0)
        pltpu.sync_copy(buf, agg_sh.at[pl.ds(rs, 2 * K)])
        pltpu.sync_copy(buf, agg_sh.at[pl.ds(rs + 2 * K, 2 * K)])
        pltpu.sync_copy(buf.at[pl.ds(0, K)],
                        agg_sh.at[pl.ds(rs + 4 * K, K)])

    def _unpack(j, half):
        for t in range(K // 16):
            v = pkv[j, pl.ds(t * 16, 16)]
            colb[half, pl.ds(t * 16, 16)] = jnp.bitwise_and(v, 16383)
            rowb[half, pl.ds(t * 16, 16)] = lax.shift_right_logical(v, 14)

    def _fire_gather(half):
        pltpu.async_copy(hs_hbm.at[colb.at[half]],
                         buf.at[pl.ds(half * K, K)], gsem)

    def _fire_scatter(half):
        pltpu.async_copy(buf.at[pl.ds(half * K, K)],
                         agg_sh.at[rowb.at[half]], ssem, add=True)

    def _wait_gather():
        pltpu.make_async_copy(hs_hbm.at[colb.at[0]],
                              buf.at[pl.ds(0, K)], gsem).wait()

    def _wait_scatter():
        pltpu.make_async_copy(buf.at[pl.ds(0, K)],
                              agg_sh.at[rowb.at[0]], ssem).wait()

    _unpack(0, 0)
    plsc.subcore_barrier()
    _fire_gather(0)

    def _body(t, carry):
        j0 = 2 * t
        _wait_gather()
        _fire_scatter(0)

        @pl.when(t >= 1)
        def _():
            _wait_scatter()

        _unpack(j0 + 1, 1)
        _fire_gather(1)

        _wait_gather()
        _fire_scatter(1)

        @pl.when(t + 1 < T)
        def _():
            _wait_scatter()
            _unpack(j0 + 2, 0)
            _fire_gather(0)

        return carry

    lax.fori_loop(0, T, _body, 0)

    for _ in range(2):
        _wait_scatter()

    plsc.subcore_barrier()
    pltpu.sync_copy(agg_sh.at[pl.ds(rs, RS)], out_hbm.at[c, pl.ds(rs, RS)])


def _embed_body(x_ref, w_ref, deg_ref, h_ref, hs_ref):
    h = jnp.dot(x_ref[...], w_ref[...], preferred_element_type=jnp.float32)
    d = deg_ref[0, :, 0:1] + deg_ref[1, :, 0:1]
    dis = lax.rsqrt(1.0 + d)
    h_ref[...] = h
    hs_ref[...] = h * dis


def _layer_body(p_ref, h_ref, deg_ref, u_ref, g_ref, b_ref, h2_ref, hs2_ref):
    d = deg_ref[0, :, 0:1] + deg_ref[1, :, 0:1]
    dis = lax.rsqrt(1.0 + d)
    agg = (p_ref[0] + p_ref[1]) * dis
    z = jnp.dot(agg, u_ref[...], preferred_element_type=jnp.float32)
    z = jnp.maximum(z, 0.0)
    z = z * (g_ref[...] * _BN_SCALE) + b_ref[...]
    z = jnp.tanh(z)
    h2 = z + h_ref[...]
    h2_ref[...] = h2
    hs2_ref[...] = h2 * dis


def _readout_body(h_ref, w0, b0, w1, b1, w2, b2, w3, b3, y_ref):
    y = h_ref[...]
    for wr, br in ((w0, b0), (w1, b1), (w2, b2)):
        y = jnp.dot(y, wr[...], preferred_element_type=jnp.float32) + br[...]
        y = jnp.maximum(y, 0.0)
    y_ref[...] = jnp.dot(y, w3[...], preferred_element_type=jnp.float32) \
        + b3[...]


def _full(shape):
    return pl.BlockSpec(shape, lambda i: (0,) * len(shape))


def _rows(shape):
    return pl.BlockSpec(shape, lambda i: (i,) + (0,) * (len(shape) - 1))


_embed_call = pl.pallas_call(
    _embed_body,
    grid=(GRID,),
    in_specs=[_rows((BN, D)), _full((D, H)), _rows((NC, BN, 16))],
    out_specs=[_rows((BN, H)), _rows((BN, H))],
    out_shape=[jax.ShapeDtypeStruct((NP, H), jnp.float32),
               jax.ShapeDtypeStruct((NP, H), jnp.float32)],
)

_layer_call = pl.pallas_call(
    _layer_body,
    grid=(GRID,),
    in_specs=[_rows((NC, BN, H)), _rows((BN, H)), _rows((NC, BN, 16)),
              _full((H, H)), _full((1, H)), _full((1, H))],
    out_specs=[_rows((BN, H)), _rows((BN, H))],
    out_shape=[jax.ShapeDtypeStruct((NP, H), jnp.float32),
               jax.ShapeDtypeStruct((NP, H), jnp.float32)],
)

_readout_call = pl.pallas_call(
    _readout_body,
    grid=(GRID,),
    in_specs=[_rows((BN, H)), _full((H, H)), _full((1, H)), _full((H, H)),
              _full((1, H)), _full((H, H)), _full((1, H)), _full((H, OUT)),
              _full((1, OUT))],
    out_specs=_rows((BN, OUT)),
    out_shape=jax.ShapeDtypeStruct((NP, OUT), jnp.float32),
)


def kernel(x, edge_index, W0, U1_W, bn1_g, bn1_b, U2_W, bn2_g, bn2_b,
           U3_W, bn3_g, bn3_b, ro_W0, ro_b0, ro_W1, ro_b1, ro_W2, ro_b2,
           ro_W3, ro_b3):
    pad_idx = N + (jnp.arange(EP - E, dtype=jnp.int32) % (NP - N))
    rowp = jnp.concatenate([edge_index[0], pad_idx])
    colp = jnp.concatenate([edge_index[1], pad_idx])
    row4 = rowp.reshape(NW, C, K)
    pk4 = (rowp * 16384 + colp).reshape(NW, C, K)
    xp = jnp.pad(x, ((0, NP - N), (0, 0)))

    deg = _deg_kernel(row4)
    h, hs = _embed_call(xp, W0, deg)
    for u_w, g, b in ((U1_W, bn1_g, bn1_b), (U2_W, bn2_g, bn2_b),
                      (U3_W, bn3_g, bn3_b)):
        p = _agg_kernel(hs, pk4)
        h, hs = _layer_call(p, h, deg, u_w, g.reshape(1, H), b.reshape(1, H))

    y = _readout_call(h, ro_W0, ro_b0.reshape(1, H), ro_W1,
                      ro_b1.reshape(1, H), ro_W2, ro_b2.reshape(1, H),
                      ro_W3, ro_b3.reshape(1, OUT))
    return y[:N]


# speedup vs baseline: 23.8500x; 1.0312x over previous
import functools

import jax
import jax.numpy as jnp
from jax import lax
from jax.experimental import pallas as pl
from jax.experimental.pallas import tpu as pltpu
from jax.experimental.pallas import tpu_sc as plsc

N = 10000
NP = 10240
E = 320000
D = 128
H = 128
OUT = 40

NC = 2
NS = 16
NW = NC * NS
K = 128
C = 80
EP = NW * C * K
T = C // 2
RS = NP // NS
BN = 2048
GRID = NP // BN

_SC_MESH = plsc.VectorSubcoreMesh(core_axis_name="c", subcore_axis_name="s")

_BN_SCALE = 1.0 / (1.0 + 1e-5) ** 0.5


@functools.partial(
    pl.kernel,
    out_type=jax.ShapeDtypeStruct((NC, NP, 16), jnp.float32),
    mesh=_SC_MESH,
    scratch_types=[
        pltpu.VMEM((C, K), jnp.int32),
        pltpu.VMEM((K, 16), jnp.float32),
        pltpu.VMEM_SHARED((NP, 16), jnp.float32),
        pltpu.SemaphoreType.DMA,
    ],
)
def _deg_kernel(row_hbm, out_hbm, rowv, onev, deg_sh, sem):
    c = lax.axis_index("c")
    s = lax.axis_index("s")
    w = c * NS + s
    pltpu.sync_copy(row_hbm.at[w], rowv)

    zero = jnp.zeros((16,), jnp.float32)

    def _zb(i, carry):
        onev[i, :] = zero
        return carry

    lax.fori_loop(0, K, _zb, 0)
    for q in range(RS // K):
        pltpu.sync_copy(onev, deg_sh.at[pl.ds(s * RS + q * K, K)])

    one = jnp.full((16,), 1.0, jnp.float32)

    def _ob(i, carry):
        onev[i, :] = one
        return carry

    lax.fori_loop(0, K, _ob, 0)
    plsc.subcore_barrier()

    def _fire(j):
        pltpu.async_copy(onev, deg_sh.at[rowv.at[j]], sem, add=True)

    for j in range(8):
        _fire(j)

    def _body(j, carry):
        @pl.when(j + 8 < C)
        def _():
            _fire(j + 8)

        pltpu.make_async_copy(onev, deg_sh.at[rowv.at[0]], sem).wait()
        return carry

    lax.fori_loop(0, C, _body, 0)
    plsc.subcore_barrier()
    pltpu.sync_copy(deg_sh.at[pl.ds(s * RS, RS)],
                    out_hbm.at[c, pl.ds(s * RS, RS)])


@functools.partial(
    pl.kernel,
    out_type=jax.ShapeDtypeStruct((NC, NP, H), jnp.float32),
    mesh=_SC_MESH,
    scratch_types=[
        pltpu.VMEM((C, K), jnp.int32),
        pltpu.VMEM((4, K // 2), jnp.int32),
        pltpu.VMEM((4, K // 2), jnp.int32),
        pltpu.VMEM((2 * K, H), jnp.float32),
        pltpu.VMEM_SHARED((NP, H), jnp.float32),
        pltpu.SemaphoreType.DMA,
        pltpu.SemaphoreType.DMA,
    ],
)
def _agg_kernel(hs_hbm, pk_hbm, out_hbm, pkv, colb, rowb, buf, agg_sh,
                gsem, ssem):
    c = lax.axis_index("c")
    s = lax.axis_index("s")
    w = c * NS + s
    pltpu.sync_copy(pk_hbm.at[w], pkv)

    rs = s * RS

    @pl.when(c == 0)
    def _():
        pltpu.sync_copy(hs_hbm.at[pl.ds(rs, RS)], agg_sh.at[pl.ds(rs, RS)])

    @pl.when(c == 1)
    def _():
        zero = jnp.zeros((16,), jnp.float32)

        def _zb(i, carry):
            r = i // (H // 16)
            l = i % (H // 16)
            buf[r, pl.ds(l * 16, 16)] = zero
            return carry

        lax.fori_loop(0, 2 * K * (H // 16), _zb, 0)
        pltpu.sync_copy(buf, agg_sh.at[pl.ds(rs, 2 * K)])
        pltpu.sync_copy(buf, agg_sh.at[pl.ds(rs + 2 * K, 2 * K)])
        pltpu.sync_copy(buf.at[pl.ds(0, K)],
                        agg_sh.at[pl.ds(rs + 4 * K, K)])

    SUB = K // 2

    def _unpack(j, p):
        for u in range(2):
            for t in range(SUB // 16):
                v = pkv[j, pl.ds(u * SUB + t * 16, 16)]
                colb[2 * p + u, pl.ds(t * 16, 16)] = \
                    jnp.bitwise_and(v, 16383)
                rowb[2 * p + u, pl.ds(t * 16, 16)] = \
                    lax.shift_right_logical(v, 14)

    def _fire_gather(slot):
        pltpu.async_copy(hs_hbm.at[colb.at[slot]],
                         buf.at[pl.ds(slot * SUB, SUB)], gsem)

    def _fire_scatter(slot):
        pltpu.async_copy(buf.at[pl.ds(slot * SUB, SUB)],
                         agg_sh.at[rowb.at[slot]], ssem, add=True)

    def _wait_gather():
        pltpu.make_async_copy(hs_hbm.at[colb.at[0]],
                              buf.at[pl.ds(0, SUB)], gsem).wait()

    def _wait_scatter():
        pltpu.make_async_copy(buf.at[pl.ds(0, SUB)],
                              agg_sh.at[rowb.at[0]], ssem).wait()

    _unpack(0, 0)
    plsc.subcore_barrier()
    _fire_gather(0)
    _fire_gather(1)

    def _body(t, carry):
        j0 = 2 * t
        _wait_gather()
        _fire_scatter(0)
        _wait_gather()
        _fire_scatter(1)

        @pl.when(t >= 1)
        def _():
            _wait_scatter()
            _wait_scatter()

        _unpack(j0 + 1, 1)
        _fire_gather(2)
        _fire_gather(3)

        _wait_gather()
        _fire_scatter(2)
        _wait_gather()
        _fire_scatter(3)

        @pl.when(t + 1 < T)
        def _():
            _wait_scatter()
            _wait_scatter()
            _unpack(j0 + 2, 0)
            _fire_gather(0)
            _fire_gather(1)

        return carry

    lax.fori_loop(0, T, _body, 0)

    for _ in range(4):
        _wait_scatter()

    plsc.subcore_barrier()
    pltpu.sync_copy(agg_sh.at[pl.ds(rs, RS)], out_hbm.at[c, pl.ds(rs, RS)])


def _mm_body(x_ref, w_ref, h_ref):
    h_ref[...] = jnp.dot(x_ref[...], w_ref[...],
                         preferred_element_type=jnp.float32)


def _scale_body(h_ref, deg_ref, hs_ref):
    d = deg_ref[0, :, 0:1] + deg_ref[1, :, 0:1]
    dis = lax.rsqrt(1.0 + d)
    hs_ref[...] = h_ref[...] * dis


def _layer_body(p_ref, h_ref, deg_ref, u_ref, g_ref, b_ref, h2_ref, hs2_ref):
    d = deg_ref[0, :, 0:1] + deg_ref[1, :, 0:1]
    dis = lax.rsqrt(1.0 + d)
    agg = (p_ref[0] + p_ref[1]) * dis
    z = jnp.dot(agg, u_ref[...], preferred_element_type=jnp.float32)
    z = jnp.maximum(z, 0.0)
    z = z * (g_ref[...] * _BN_SCALE) + b_ref[...]
    z = jnp.tanh(z)
    h2 = z + h_ref[...]
    h2_ref[...] = h2
    hs2_ref[...] = h2 * dis


def _final_body(p_ref, h_ref, deg_ref, u_ref, g_ref, b_ref,
                w0, b0, w1, b1, w2, b2, w3, b3, y_ref):
    d = deg_ref[0, :, 0:1] + deg_ref[1, :, 0:1]
    dis = lax.rsqrt(1.0 + d)
    agg = (p_ref[0] + p_ref[1]) * dis
    z = jnp.dot(agg, u_ref[...], preferred_element_type=jnp.float32)
    z = jnp.maximum(z, 0.0)
    z = z * (g_ref[...] * _BN_SCALE) + b_ref[...]
    y = jnp.tanh(z) + h_ref[...]
    for wr, br in ((w0, b0), (w1, b1), (w2, b2)):
        y = jnp.dot(y, wr[...], preferred_element_type=jnp.float32) + br[...]
        y = jnp.maximum(y, 0.0)
    y_ref[...] = jnp.dot(y, w3[...], preferred_element_type=jnp.float32) \
        + b3[...]


def _full(shape):
    return pl.BlockSpec(shape, lambda i: (0,) * len(shape))


def _rows(shape):
    return pl.BlockSpec(shape, lambda i: (i,) + (0,) * (len(shape) - 1))


_mm_call = pl.pallas_call(
    _mm_body,
    grid=(GRID,),
    in_specs=[_rows((BN, D)), _full((D, H))],
    out_specs=_rows((BN, H)),
    out_shape=jax.ShapeDtypeStruct((NP, H), jnp.float32),
)

_scale_call = pl.pallas_call(
    _scale_body,
    grid=(GRID,),
    in_specs=[_rows((BN, H)), _rows((NC, BN, 16))],
    out_specs=_rows((BN, H)),
    out_shape=jax.ShapeDtypeStruct((NP, H), jnp.float32),
)

_layer_call = pl.pallas_call(
    _layer_body,
    grid=(GRID,),
    in_specs=[_rows((NC, BN, H)), _rows((BN, H)), _rows((NC, BN, 16)),
              _full((H, H)), _full((1, H)), _full((1, H))],
    out_specs=[_rows((BN, H)), _rows((BN, H))],
    out_shape=[jax.ShapeDtypeStruct((NP, H), jnp.float32),
               jax.ShapeDtypeStruct((NP, H), jnp.float32)],
)

_final_call = pl.pallas_call(
    _final_body,
    grid=(GRID,),
    in_specs=[_rows((NC, BN, H)), _rows((BN, H)), _rows((NC, BN, 16)),
              _full((H, H)), _full((1, H)), _full((1, H)),
              _full((H, H)), _full((1, H)), _full((H, H)),
              _full((1, H)), _full((H, H)), _full((1, H)), _full((H, OUT)),
              _full((1, OUT))],
    out_specs=_rows((BN, OUT)),
    out_shape=jax.ShapeDtypeStruct((NP, OUT), jnp.float32),
)


def kernel(x, edge_index, W0, U1_W, bn1_g, bn1_b, U2_W, bn2_g, bn2_b,
           U3_W, bn3_g, bn3_b, ro_W0, ro_b0, ro_W1, ro_b1, ro_W2, ro_b2,
           ro_W3, ro_b3):
    pad_idx = N + (jnp.arange(EP - E, dtype=jnp.int32) % (NP - N))
    rowp = jnp.concatenate([edge_index[0], pad_idx])
    colp = jnp.concatenate([edge_index[1], pad_idx])
    row4 = rowp.reshape(NW, C, K)
    pk4 = (rowp * 16384 + colp).reshape(NW, C, K)
    xp = jnp.pad(x, ((0, NP - N), (0, 0)))

    deg = _deg_kernel(row4)
    h = _mm_call(xp, W0)
    hs = _scale_call(h, deg)
    for u_w, g, b in ((U1_W, bn1_g, bn1_b), (U2_W, bn2_g, bn2_b)):
        p = _agg_kernel(hs, pk4)
        h, hs = _layer_call(p, h, deg, u_w, g.reshape(1, H), b.reshape(1, H))

    p = _agg_kernel(hs, pk4)
    y = _final_call(p, h, deg, U3_W, bn3_g.reshape(1, H), bn3_b.reshape(1, H),
                    ro_W0, ro_b0.reshape(1, H), ro_W1, ro_b1.reshape(1, H),
                    ro_W2, ro_b2.reshape(1, H), ro_W3, ro_b3.reshape(1, OUT))
    return y[:N]
